# Initial kernel scaffold; baseline (speedup 1.0000x reference)
import jax
import jax.numpy as jnp
from jax.experimental import pallas as pl


def kernel(x, indices, W, lora_a, lora_b):
    raise NotImplementedError("write your pallas kernel here")


# speedup vs baseline: 1.1475x; 1.1475x over previous
import jax
import jax.numpy as jnp
from jax.experimental import pallas as pl
from jax.experimental.pallas import tpu as pltpu

_E = 8
_D = 1024
_O = 1024
_B = 2048
_SCALE = 20.0
_TB = 128


def _fused_body(idx_ref, x_ref, w_ref, a_ref, lb_ref, out_ref):
    x = x_ref[...]
    idx = idx_ref[...]
    y = jnp.zeros((_TB, _O), jnp.float32)
    for e in range(_E):
        ye = jax.lax.dot_general(
            x, w_ref[e], (((1,), (1,)), ((), ())),
            preferred_element_type=jnp.float32)
        y = jnp.where(idx == e, ye, y)
    a2 = _SCALE * a_ref[...] * jnp.sum(lb_ref[...], axis=1, keepdims=True)
    sz = jax.lax.dot_general(
        x, a2, (((1,), (1,)), ((), ())),
        preferred_element_type=jnp.float32)
    for e in range(_E):
        out_ref[:, e, :] = y + sz[:, e:e + 1]


def kernel(x, indices, W, lora_a, lora_b):
    a_mat = lora_a.reshape(_E, _D)
    lb_mat = lora_b.reshape(_E, _O)
    grid = (_B // _TB,)
    out = pl.pallas_call(
        _fused_body,
        grid=grid,
        in_specs=[
            pl.BlockSpec((_TB, 1), lambda i: (i, 0)),
            pl.BlockSpec((_TB, _D), lambda i: (i, 0)),
            pl.BlockSpec((_E, _O, _D), lambda i: (0, 0, 0)),
            pl.BlockSpec((_E, _D), lambda i: (0, 0)),
            pl.BlockSpec((_E, _O), lambda i: (0, 0)),
        ],
        out_specs=pl.BlockSpec((_TB, _E, _O), lambda i: (i, 0, 0)),
        out_shape=jax.ShapeDtypeStruct((_B, _E, _O), jnp.float32),
        compiler_params=pltpu.CompilerParams(
            dimension_semantics=("arbitrary",),
        ),
    )(indices, x, W, a_mat, lb_mat)
    return out
